# baseline (speedup 1.0000x reference)
import functools

import jax
import jax.numpy as jnp
from jax import lax
from jax.experimental import pallas as pl
from jax.experimental.pallas import tpu as pltpu
from jax.experimental.pallas import tpu_sc as plsc

_NUM_CODEBOOKS = 8
_VOCAB = 1024
_D_IN = 64
_D_OUT = 64
_B, _T = 16, 2048
_N = _B * _T * _NUM_CODEBOOKS
_LANES = 128
_TT = _T // _LANES
_ROWS = _N // _LANES
_RING = 4


def _tc_prep_body(tokens_ref, table_ref, w_ref, b_ref, idx_ref, p_ref):
    x = tokens_ref[...]
    for c in range(_NUM_CODEBOOKS):
        idx_ref[:, c] = x[:, :, c].reshape(_B, _TT, _LANES) + c * _VOCAB
    p_ref[...] = jnp.dot(table_ref[...], w_ref[...],
                         preferred_element_type=jnp.float32) + b_ref[...]


def _tc_prep(in_tokens, emb_table, W, b2d):
    return pl.pallas_call(
        _tc_prep_body,
        out_shape=[
            jax.ShapeDtypeStruct((_B, _NUM_CODEBOOKS, _TT, _LANES), jnp.int32),
            jax.ShapeDtypeStruct((_NUM_CODEBOOKS * _VOCAB, _D_OUT), jnp.float32),
        ],
    )(in_tokens, emb_table, W, b2d)


def _make_sc_gather(nw, chunks_per_w, num_cores):
    mesh = plsc.VectorSubcoreMesh(core_axis_name="c", subcore_axis_name="s")

    @functools.partial(
        pl.kernel,
        mesh=mesh,
        compiler_params=pltpu.CompilerParams(use_tc_tiling_on_sc=False),
        out_type=jax.ShapeDtypeStruct((_N, 2 * _D_OUT), jnp.float32),
        scratch_types=[
            pltpu.VMEM((chunks_per_w, _LANES), jnp.int32),
            pltpu.VMEM((_RING, _LANES, _D_OUT), jnp.float32),
            pltpu.SemaphoreType.DMA,
            pltpu.SemaphoreType.DMA,
        ],
    )
    def sc_gather(p_hbm, idx_hbm, out_hbm, idx_v, bufs, gsem, wsem):
        wid = lax.axis_index("s") * num_cores + lax.axis_index("c")
        base = wid * (chunks_per_w * _LANES)
        pltpu.sync_copy(idx_hbm.at[wid], idx_v)

        for q in range(_RING):
            pltpu.async_copy(p_hbm.at[idx_v.at[q]], bufs.at[q], gsem)

        n_groups = chunks_per_w // _RING

        def body(j, _):
            c0 = _RING * j
            for q in range(_RING):
                pltpu.make_async_copy(
                    p_hbm.at[idx_v.at[c0 + q]], bufs.at[q], gsem).wait()
                pltpu.async_copy(
                    bufs.at[q],
                    out_hbm.at[pl.ds(base + (c0 + q) * _LANES, _LANES),
                               pl.ds(0, _D_OUT)],
                    wsem)

            @pl.when(j < n_groups - 1)
            def _():
                for q in range(_RING):
                    pltpu.make_async_copy(
                        bufs.at[q],
                        out_hbm.at[pl.ds(base + (c0 + q) * _LANES, _LANES),
                                   pl.ds(0, _D_OUT)],
                        wsem).wait()
                    pltpu.async_copy(
                        p_hbm.at[idx_v.at[c0 + _RING + q]], bufs.at[q], gsem)
            return 0

        lax.fori_loop(0, n_groups, body, 0)

        last = chunks_per_w - _RING
        for q in range(_RING):
            pltpu.make_async_copy(
                bufs.at[q],
                out_hbm.at[pl.ds(base + (last + q) * _LANES, _LANES),
                           pl.ds(0, _D_OUT)],
                wsem).wait()

    return sc_gather


def _transpose_body(a_ref, x_ref):
    x_ref[0, 0] = jnp.transpose(a_ref[...][:, :_D_OUT])


def _transpose_finisher(a2d):
    return pl.pallas_call(
        _transpose_body,
        grid=(_B, _NUM_CODEBOOKS),
        in_specs=[pl.BlockSpec((_T, 2 * _D_OUT),
                               lambda i, j: (i * _NUM_CODEBOOKS + j, 0))],
        out_specs=pl.BlockSpec((1, 1, _D_OUT, _T),
                               lambda i, j: (i, j, 0, 0)),
        out_shape=jax.ShapeDtypeStruct((_B, _NUM_CODEBOOKS, _D_OUT, _T),
                                       jnp.float32),
    )(a2d)


def kernel(in_tokens, emb_table, W, b):
    info = plsc.get_sparse_core_info()
    nw = info.num_cores * info.num_subcores
    chunks_per_w = _ROWS // nw
    idx4, proj = _tc_prep(in_tokens, emb_table, W, b.reshape(1, _D_OUT))
    idx3d = idx4.reshape(nw, chunks_per_w, _LANES)
    sc_gather = _make_sc_gather(nw, chunks_per_w, info.num_cores)
    a2d = sc_gather(proj, idx3d)
    x = _transpose_finisher(a2d)
    return jnp.transpose(x, (0, 3, 1, 2))


# speedup vs baseline: 1.2084x; 1.2084x over previous
import functools

import jax
import jax.numpy as jnp
from jax import lax
from jax.experimental import pallas as pl
from jax.experimental.pallas import tpu as pltpu
from jax.experimental.pallas import tpu_sc as plsc

_NUM_CODEBOOKS = 8
_VOCAB = 1024
_D_IN = 64
_D_OUT = 64
_B, _T = 16, 2048
_N = _B * _T * _NUM_CODEBOOKS
_LANES = 128
_TT = _T // _LANES
_ROWS = _N // _LANES
_RING = 4


def _tc_prep_body(tokens_ref, table_ref, w_ref, b_ref, idx_ref, p_ref):
    x = tokens_ref[...]
    offs = lax.broadcasted_iota(jnp.int32, (_B, _NUM_CODEBOOKS, _TT, _LANES), 1) * _VOCAB
    idx_ref[...] = x.reshape(_B, _NUM_CODEBOOKS, _TT, _LANES) + offs
    p_ref[...] = jnp.dot(table_ref[...], w_ref[...],
                         preferred_element_type=jnp.float32) + b_ref[...]


def _tc_prep(in_tokens, emb_table, W, b2d):
    return pl.pallas_call(
        _tc_prep_body,
        out_shape=[
            jax.ShapeDtypeStruct((_B, _NUM_CODEBOOKS, _TT, _LANES), jnp.int32),
            jax.ShapeDtypeStruct((_NUM_CODEBOOKS * _VOCAB, _D_OUT), jnp.float32),
        ],
    )(in_tokens, emb_table, W, b2d)


def _make_sc_gather(nw, chunks_per_w, num_cores):
    mesh = plsc.VectorSubcoreMesh(core_axis_name="c", subcore_axis_name="s")

    @functools.partial(
        pl.kernel,
        mesh=mesh,
        compiler_params=pltpu.CompilerParams(use_tc_tiling_on_sc=False),
        out_type=jax.ShapeDtypeStruct((_N, 2 * _D_OUT), jnp.float32),
        scratch_types=[
            pltpu.VMEM((chunks_per_w, _LANES), jnp.int32),
            pltpu.VMEM((_RING, _LANES, _D_OUT), jnp.float32),
            pltpu.SemaphoreType.DMA,
            pltpu.SemaphoreType.DMA,
        ],
    )
    def sc_gather(p_hbm, idx_hbm, out_hbm, idx_v, bufs, gsem, wsem):
        wid = lax.axis_index("s") * num_cores + lax.axis_index("c")
        base = wid * (chunks_per_w * _LANES)
        pltpu.sync_copy(idx_hbm.at[wid], idx_v)

        for q in range(_RING):
            pltpu.async_copy(p_hbm.at[idx_v.at[q]], bufs.at[q], gsem)

        n_groups = chunks_per_w // _RING

        def body(j, _):
            c0 = _RING * j
            for q in range(_RING):
                pltpu.make_async_copy(
                    p_hbm.at[idx_v.at[c0 + q]], bufs.at[q], gsem).wait()
                pltpu.async_copy(
                    bufs.at[q],
                    out_hbm.at[pl.ds(base + (c0 + q) * _LANES, _LANES),
                               pl.ds(0, _D_OUT)],
                    wsem)

            @pl.when(j < n_groups - 1)
            def _():
                for q in range(_RING):
                    pltpu.make_async_copy(
                        bufs.at[q],
                        out_hbm.at[pl.ds(base + (c0 + q) * _LANES, _LANES),
                                   pl.ds(0, _D_OUT)],
                        wsem).wait()
                    pltpu.async_copy(
                        p_hbm.at[idx_v.at[c0 + _RING + q]], bufs.at[q], gsem)
            return 0

        lax.fori_loop(0, n_groups, body, 0)

        last = chunks_per_w - _RING
        for q in range(_RING):
            pltpu.make_async_copy(
                bufs.at[q],
                out_hbm.at[pl.ds(base + (last + q) * _LANES, _LANES),
                           pl.ds(0, _D_OUT)],
                wsem).wait()

    return sc_gather


def _transpose_body(a_ref, x_ref):
    x_ref[0, 0] = jnp.transpose(a_ref[...][:, :_D_OUT])


def _transpose_finisher(a2d):
    return pl.pallas_call(
        _transpose_body,
        grid=(_B, _NUM_CODEBOOKS),
        in_specs=[pl.BlockSpec((_T, 2 * _D_OUT),
                               lambda i, j: (i * _NUM_CODEBOOKS + j, 0))],
        out_specs=pl.BlockSpec((1, 1, _D_OUT, _T),
                               lambda i, j: (i, j, 0, 0)),
        out_shape=jax.ShapeDtypeStruct((_B, _NUM_CODEBOOKS, _D_OUT, _T),
                                       jnp.float32),
    )(a2d)


def kernel(in_tokens, emb_table, W, b):
    info = plsc.get_sparse_core_info()
    nw = info.num_cores * info.num_subcores
    chunks_per_w = _ROWS // nw
    tokens_t = jnp.transpose(in_tokens, (0, 2, 1))
    idx4, proj = _tc_prep(tokens_t, emb_table, W, b.reshape(1, _D_OUT))
    idx3d = idx4.reshape(nw, chunks_per_w, _LANES)
    sc_gather = _make_sc_gather(nw, chunks_per_w, info.num_cores)
    a2d = sc_gather(proj, idx3d)
    x = _transpose_finisher(a2d)
    return jnp.transpose(x, (0, 3, 1, 2))


# speedup vs baseline: 1.2114x; 1.0025x over previous
import functools

import jax
import jax.numpy as jnp
from jax import lax
from jax.experimental import pallas as pl
from jax.experimental.pallas import tpu as pltpu
from jax.experimental.pallas import tpu_sc as plsc

_NUM_CODEBOOKS = 8
_VOCAB = 1024
_D_IN = 64
_D_OUT = 64
_B, _T = 16, 2048
_N = _B * _T * _NUM_CODEBOOKS
_LANES = 128
_TT = _T // _LANES
_ROWS = _N // _LANES
_RING = 4


def _tc_prep_body(tokens_ref, table_ref, w_ref, b_ref, idx_ref, p_ref):
    x = tokens_ref[...]
    offs = lax.broadcasted_iota(jnp.int32, (_B, _NUM_CODEBOOKS, _TT, _LANES), 1) * _VOCAB
    idx_ref[...] = x.reshape(_B, _NUM_CODEBOOKS, _TT, _LANES) + offs
    p_ref[...] = jnp.dot(table_ref[...], w_ref[...],
                         preferred_element_type=jnp.float32) + b_ref[...]


def _tc_prep(in_tokens, emb_table, W, b2d):
    return pl.pallas_call(
        _tc_prep_body,
        out_shape=[
            jax.ShapeDtypeStruct((_B, _NUM_CODEBOOKS, _TT, _LANES), jnp.int32),
            jax.ShapeDtypeStruct((_NUM_CODEBOOKS * _VOCAB, _D_OUT), jnp.float32),
        ],
    )(in_tokens, emb_table, W, b2d)


def _make_sc_gather(nw, chunks_per_w, num_cores):
    mesh = plsc.VectorSubcoreMesh(core_axis_name="c", subcore_axis_name="s")

    @functools.partial(
        pl.kernel,
        mesh=mesh,
        compiler_params=pltpu.CompilerParams(use_tc_tiling_on_sc=False),
        out_type=jax.ShapeDtypeStruct((_N, 2 * _D_OUT), jnp.float32),
        scratch_types=[
            pltpu.VMEM((chunks_per_w, _LANES), jnp.int32),
            pltpu.VMEM((_RING, _LANES, _D_OUT), jnp.float32),
            pltpu.SemaphoreType.DMA,
            pltpu.SemaphoreType.DMA,
        ],
    )
    def sc_gather(p_hbm, idx_hbm, out_hbm, idx_v, bufs, gsem, wsem):
        wid = lax.axis_index("s") * num_cores + lax.axis_index("c")
        base = wid * (chunks_per_w * _LANES)
        pltpu.sync_copy(idx_hbm.at[wid], idx_v)

        for q in range(_RING):
            pltpu.async_copy(p_hbm.at[idx_v.at[q]], bufs.at[q], gsem)

        n_groups = chunks_per_w // _RING

        def body(j, _):
            c0 = _RING * j
            for q in range(_RING):
                pltpu.make_async_copy(
                    p_hbm.at[idx_v.at[c0 + q]], bufs.at[q], gsem).wait()
                pltpu.async_copy(
                    bufs.at[q],
                    out_hbm.at[pl.ds(base + (c0 + q) * _LANES, _LANES),
                               pl.ds(0, _D_OUT)],
                    wsem)

            @pl.when(j < n_groups - 1)
            def _():
                for q in range(_RING):
                    pltpu.make_async_copy(
                        bufs.at[q],
                        out_hbm.at[pl.ds(base + (c0 + q) * _LANES, _LANES),
                                   pl.ds(0, _D_OUT)],
                        wsem).wait()
                    pltpu.async_copy(
                        p_hbm.at[idx_v.at[c0 + _RING + q]], bufs.at[q], gsem)
            return 0

        lax.fori_loop(0, n_groups, body, 0)

        last = chunks_per_w - _RING
        for q in range(_RING):
            pltpu.make_async_copy(
                bufs.at[q],
                out_hbm.at[pl.ds(base + (last + q) * _LANES, _LANES),
                           pl.ds(0, _D_OUT)],
                wsem).wait()

    return sc_gather


def _transpose_body(a_ref, x_ref):
    xt = jnp.transpose(a_ref[...])
    x_ref[0, 0] = xt[:_D_OUT]


def _transpose_finisher(a2d):
    return pl.pallas_call(
        _transpose_body,
        grid=(_B, _NUM_CODEBOOKS),
        in_specs=[pl.BlockSpec((_T, 2 * _D_OUT),
                               lambda i, j: (i * _NUM_CODEBOOKS + j, 0))],
        out_specs=pl.BlockSpec((1, 1, _D_OUT, _T),
                               lambda i, j: (i, j, 0, 0)),
        out_shape=jax.ShapeDtypeStruct((_B, _NUM_CODEBOOKS, _D_OUT, _T),
                                       jnp.float32),
    )(a2d)


def kernel(in_tokens, emb_table, W, b):
    info = plsc.get_sparse_core_info()
    nw = info.num_cores * info.num_subcores
    chunks_per_w = _ROWS // nw
    tokens_t = jnp.transpose(in_tokens, (0, 2, 1))
    idx4, proj = _tc_prep(tokens_t, emb_table, W, b.reshape(1, _D_OUT))
    idx3d = idx4.reshape(nw, chunks_per_w, _LANES)
    sc_gather = _make_sc_gather(nw, chunks_per_w, info.num_cores)
    a2d = sc_gather(proj, idx3d)
    x = _transpose_finisher(a2d)
    return jnp.transpose(x, (0, 3, 1, 2))
